# baseline (device time: 27197 ns/iter reference)
import jax
import jax.numpy as jnp
from jax import lax
from jax.experimental import pallas as pl
from jax.experimental.pallas import tpu as pltpu

K = 32
HALF_ROWS = 512
N_BLOCKS = 4
BLOCK_ROWS = HALF_ROWS // N_BLOCKS
NEG = float("-inf")


def _topk_desc(v):
    cols = []
    for i in range(K):
        m = jnp.max(v, axis=1, keepdims=True)
        cols.append(m)
        if i < K - 1:
            v = jnp.where(v == m, NEG, v)
    return jnp.concatenate(cols, axis=1)


def _top2_strided(v, n_groups):
    w = v.shape[1] // n_groups
    chunks = [v[:, i * w:(i + 1) * w] for i in range(n_groups)]
    m1 = chunks[0]
    for c in chunks[1:]:
        m1 = jnp.maximum(m1, c)
    m2 = None
    for c in chunks:
        cm = jnp.where(c == m1, NEG, c)
        m2 = cm if m2 is None else jnp.maximum(m2, cm)
    return m1, m2


def _candidates(blk):
    m1, m2 = _top2_strided(blk, 16)
    m2s = jnp.concatenate([m2[:, -64:], m2[:, :-64]], axis=1)
    c1 = jnp.concatenate([m1, m2s], axis=1)
    n1, n2 = _top2_strided(c1, 8)
    return jnp.concatenate([n1, n2], axis=1)


def kernel(x):
    rows, n_local = x.shape

    def body(x_hbm, out_ref, xloc, gat, copy_sems, send_sems, recv_sems):
        my_x = lax.axis_index("x")
        my_y = lax.axis_index("y")
        peers = (
            (my_x, 1 - my_y),
            (1 - my_x, my_y),
            (1 - my_x, 1 - my_y),
        )

        barrier_sem = pltpu.get_barrier_semaphore()
        for nbr in peers:
            pl.semaphore_signal(
                barrier_sem, inc=1,
                device_id=nbr, device_id_type=pl.DeviceIdType.MESH,
            )
        pl.semaphore_wait(barrier_sem, 3)

        row0 = my_x * HALF_ROWS
        cps = []
        for i in range(N_BLOCKS):
            cp = pltpu.make_async_copy(
                x_hbm.at[pl.ds(row0 + i * BLOCK_ROWS, BLOCK_ROWS), :],
                xloc.at[pl.ds(i * BLOCK_ROWS, BLOCK_ROWS), :],
                copy_sems.at[i],
            )
            cp.start()
            cps.append(cp)

        cand_blocks = []
        for i in range(N_BLOCKS):
            cps[i].wait()
            blk = xloc[pl.ds(i * BLOCK_ROWS, BLOCK_ROWS), :]
            cand_blocks.append(_candidates(blk))
        cand = jnp.concatenate(cand_blocks, axis=0)

        topk_local = _topk_desc(cand)

        slot = my_x * 2 + my_y
        gat[slot] = topk_local
        rdmas = []
        for j, nbr in enumerate(peers):
            r = pltpu.make_async_remote_copy(
                src_ref=gat.at[slot], dst_ref=gat.at[slot],
                send_sem=send_sems.at[j], recv_sem=recv_sems.at[j],
                device_id=nbr, device_id_type=pl.DeviceIdType.MESH,
            )
            r.start()
            rdmas.append(r)
        for r in rdmas:
            r.wait()

        cand_all = jnp.concatenate(
            [
                jnp.concatenate([gat[0], gat[1]], axis=1),
                jnp.concatenate([gat[2], gat[3]], axis=1),
            ],
            axis=0,
        )
        out_ref[:, :] = _topk_desc(cand_all)

    return pl.pallas_call(
        body,
        out_shape=jax.ShapeDtypeStruct((rows, K), jnp.float32),
        in_specs=[pl.BlockSpec(memory_space=pl.ANY)],
        out_specs=pl.BlockSpec(memory_space=pltpu.VMEM),
        scratch_shapes=[
            pltpu.VMEM((HALF_ROWS, n_local), jnp.float32),
            pltpu.VMEM((4, HALF_ROWS, K), jnp.float32),
            pltpu.SemaphoreType.DMA((N_BLOCKS,)),
            pltpu.SemaphoreType.DMA((3,)),
            pltpu.SemaphoreType.DMA((3,)),
        ],
        compiler_params=pltpu.CompilerParams(
            collective_id=0,
            vmem_limit_bytes=100 * 1024 * 1024,
        ),
    )(x)


# device time: 22727 ns/iter; 1.1967x vs baseline; 1.1967x over previous
import jax
import jax.numpy as jnp
from jax import lax
from jax.experimental import pallas as pl
from jax.experimental.pallas import tpu as pltpu

K = 32
HALF_ROWS = 512
N_BLOCKS = 4
SB_BLOCKS = (2, 2)
N_SUPER = len(SB_BLOCKS)
BLOCK_ROWS = HALF_ROWS // N_BLOCKS
NEG = float("-inf")


def _topk_desc(v):
    cols = []
    for i in range(K):
        m = jnp.max(v, axis=1, keepdims=True)
        cols.append(m)
        if i < K - 1:
            v = jnp.where(v == m, NEG, v)
    return jnp.concatenate(cols, axis=1)


def _top2_strided(v, n_groups):
    w = v.shape[1] // n_groups
    chunks = [v[:, i * w:(i + 1) * w] for i in range(n_groups)]
    m1 = chunks[0]
    for c in chunks[1:]:
        m1 = jnp.maximum(m1, c)
    m2 = None
    for c in chunks:
        cm = jnp.where(c == m1, NEG, c)
        m2 = cm if m2 is None else jnp.maximum(m2, cm)
    return m1, m2


def _merge_sorted_topk(a, b):
    rb = b
    for j in (16, 8, 4, 2, 1):
        mask = (lax.broadcasted_iota(jnp.int32, (1, K), 1) % (2 * j)) < j
        rb = jnp.where(mask, jnp.roll(rb, -j, axis=1), jnp.roll(rb, j, axis=1))
    c = jnp.maximum(a, rb)
    for j in (16, 8, 4, 2, 1):
        mask = (lax.broadcasted_iota(jnp.int32, (1, K), 1) % (2 * j)) < j
        up = jnp.maximum(c, jnp.roll(c, -j, axis=1))
        dn = jnp.minimum(c, jnp.roll(c, j, axis=1))
        c = jnp.where(mask, up, dn)
    return c


def _candidates(blk):
    m1, m2 = _top2_strided(blk, 16)
    m2s = jnp.concatenate([m2[:, -64:], m2[:, :-64]], axis=1)
    c1 = jnp.concatenate([m1, m2s], axis=1)
    n1, n2 = _top2_strided(c1, 8)
    return jnp.concatenate([n1, n2], axis=1)


def kernel(x):
    rows, n_local = x.shape

    def body(x_hbm, out_ref, xloc, mine, ybuf, xbuf, dbuf,
             copy_sems, send_sems, recv_sems):
        my_x = lax.axis_index("x")
        my_y = lax.axis_index("y")
        peers = (
            (my_x, 1 - my_y),
            (1 - my_x, my_y),
            (1 - my_x, 1 - my_y),
        )

        row0 = my_x * HALF_ROWS
        cps = []
        for i in range(N_BLOCKS):
            cp = pltpu.make_async_copy(
                x_hbm.at[pl.ds(row0 + i * BLOCK_ROWS, BLOCK_ROWS), :],
                xloc.at[pl.ds(i * BLOCK_ROWS, BLOCK_ROWS), :],
                copy_sems.at[i],
            )
            cp.start()
            cps.append(cp)

        barrier_sem = pltpu.get_barrier_semaphore()
        for nbr in peers:
            pl.semaphore_signal(
                barrier_sem, inc=1,
                device_id=nbr, device_id_type=pl.DeviceIdType.MESH,
            )
        pl.semaphore_wait(barrier_sem, 3)

        sb_starts = [sum(SB_BLOCKS[:s]) for s in range(N_SUPER)]
        topk_sbs = []
        rdmas = []
        for sb in range(N_SUPER):
            cand_blocks = []
            for i in range(sb_starts[sb], sb_starts[sb] + SB_BLOCKS[sb]):
                cps[i].wait()
                blk = xloc[pl.ds(i * BLOCK_ROWS, BLOCK_ROWS), :]
                cand_blocks.append(_candidates(blk))
            cand = jnp.concatenate(cand_blocks, axis=0)
            topk_sb = _topk_desc(cand)
            topk_sbs.append(topk_sb)
            sb_rows = SB_BLOCKS[sb] * BLOCK_ROWS
            r0 = sb_starts[sb] * BLOCK_ROWS
            mine[pl.ds(r0, sb_rows), :] = topk_sb
            for j, (nbr, dst) in (
                (2, (peers[2], dbuf)),
                (0, (peers[0], ybuf)),
                (1, (peers[1], xbuf)),
            ):
                r = pltpu.make_async_remote_copy(
                    src_ref=mine.at[pl.ds(r0, sb_rows), :],
                    dst_ref=dst.at[pl.ds(r0, sb_rows), :],
                    send_sem=send_sems.at[j, sb], recv_sem=recv_sems.at[j, sb],
                    device_id=nbr, device_id_type=pl.DeviceIdType.MESH,
                )
                r.start()
                rdmas.append(r)
        for sb in range(N_SUPER):
            for r in rdmas[3 * sb:3 * (sb + 1)]:
                r.wait()
            sb_rows = SB_BLOCKS[sb] * BLOCK_ROWS
            r0 = sb_starts[sb] * BLOCK_ROWS
            sl = pl.ds(r0, sb_rows)
            a = jnp.concatenate([topk_sbs[sb], xbuf[sl, :]], axis=0)
            b = jnp.concatenate([ybuf[sl, :], dbuf[sl, :]], axis=0)
            merged = _merge_sorted_topk(a, b)
            out_ref[pl.ds(row0 + r0, sb_rows), :] = merged[:sb_rows, :]
            out_ref[pl.ds((1 - my_x) * HALF_ROWS + r0, sb_rows), :] = (
                merged[sb_rows:, :]
            )

    return pl.pallas_call(
        body,
        out_shape=jax.ShapeDtypeStruct((rows, K), jnp.float32),
        in_specs=[pl.BlockSpec(memory_space=pl.ANY)],
        out_specs=pl.BlockSpec(memory_space=pltpu.VMEM),
        scratch_shapes=[
            pltpu.VMEM((HALF_ROWS, n_local), jnp.float32),
            pltpu.VMEM((HALF_ROWS, K), jnp.float32),
            pltpu.VMEM((HALF_ROWS, K), jnp.float32),
            pltpu.VMEM((HALF_ROWS, K), jnp.float32),
            pltpu.VMEM((HALF_ROWS, K), jnp.float32),
            pltpu.SemaphoreType.DMA((N_BLOCKS,)),
            pltpu.SemaphoreType.DMA((3, N_SUPER)),
            pltpu.SemaphoreType.DMA((3, N_SUPER)),
        ],
        compiler_params=pltpu.CompilerParams(
            collective_id=0,
            vmem_limit_bytes=100 * 1024 * 1024,
        ),
    )(x)
